# Initial kernel scaffold; baseline (speedup 1.0000x reference)
import jax
import jax.numpy as jnp
from jax.experimental import pallas as pl


def kernel(src, seg, word_table, position_table, segment_table, ln_gamma, ln_beta):
    raise NotImplementedError("write your pallas kernel here")


# speedup vs baseline: 1.1745x; 1.1745x over previous
import functools

import jax
import jax.numpy as jnp
from jax import lax
from jax.experimental import pallas as pl
from jax.experimental.pallas import tpu as pltpu
from jax.experimental.pallas import tpu_sc as plsc

B, L, V, D = 1024, 200, 1000000, 64
N = B * L


@functools.cache
def _make_sc_gather():
    info = plsc.get_sparse_core_info()
    NC, NS = info.num_cores, info.num_subcores
    NW = NC * NS
    per_w = N // NW
    CH = 800
    NCH = per_w // CH

    mesh = plsc.VectorSubcoreMesh(core_axis_name="c", subcore_axis_name="s")

    @functools.partial(
        pl.kernel,
        mesh=mesh,
        out_type=jax.ShapeDtypeStruct((N, D), jnp.float32),
        scratch_types=[
            pltpu.VMEM((CH,), jnp.int32),
            pltpu.VMEM((CH, D), jnp.float32),
            pltpu.SemaphoreType.DMA,
        ],
        compiler_params=pltpu.CompilerParams(use_tc_tiling_on_sc=False),
    )
    def gather_kernel(idx_hbm, table_hbm, out_hbm, idx_v, rows_v, sem):
        wid = lax.axis_index("s") * NC + lax.axis_index("c")
        base = wid * per_w

        def body(c, carry):
            off = base + c * CH
            pltpu.sync_copy(idx_hbm.at[pl.ds(off, CH)], idx_v)
            pltpu.async_copy(table_hbm.at[idx_v], rows_v, sem).wait()
            pltpu.sync_copy(rows_v, out_hbm.at[pl.ds(off, CH)])
            return carry

        lax.fori_loop(0, NCH, body, 0)

    return gather_kernel


_BB = 32


def _ln_body(g_ref, seg_ref, pos_ref, par_ref, o_ref):
    x = g_ref[...]
    seg = seg_ref[...]
    pos = pos_ref[...]
    par = par_ref[...]

    s0 = par[0:1, :].reshape(1, 1, D)
    d1 = (par[1:2, :] - par[0:1, :]).reshape(1, 1, D)
    d2 = (par[2:3, :] - par[1:2, :]).reshape(1, 1, D)
    gam = par[3:4, :].reshape(1, 1, D)
    bet = par[4:5, :].reshape(1, 1, D)

    m1 = (seg >= 1).astype(jnp.float32)
    m2 = (seg >= 2).astype(jnp.float32)
    x = x + pos + s0 + m1 * d1 + m2 * d2

    mu = jnp.mean(x, axis=-1, keepdims=True)
    xc = x - mu
    var = jnp.mean(xc * xc, axis=-1, keepdims=True)
    o_ref[...] = xc * lax.rsqrt(var + 1e-6) * gam + bet


def _ln_call(gathered, seg3, pos, params):
    return pl.pallas_call(
        _ln_body,
        grid=(B // _BB,),
        in_specs=[
            pl.BlockSpec((_BB, L, D), lambda i: (i, 0, 0)),
            pl.BlockSpec((_BB, L, 1), lambda i: (i, 0, 0)),
            pl.BlockSpec((1, L, D), lambda i: (0, 0, 0)),
            pl.BlockSpec((8, D), lambda i: (0, 0)),
        ],
        out_specs=pl.BlockSpec((_BB, L, D), lambda i: (i, 0, 0)),
        out_shape=jax.ShapeDtypeStruct((B, L, D), jnp.float32),
    )(gathered, seg3, pos, params)


def kernel(src, seg, word_table, position_table, segment_table, ln_gamma, ln_beta):
    idx = src.reshape(N).astype(jnp.int32)
    gathered = _make_sc_gather()(idx, word_table)
    gathered = gathered.reshape(B, L, D)

    seg3 = seg.reshape(B, L, 1).astype(jnp.int32)
    pos = position_table[:L].reshape(1, L, D)
    params = jnp.concatenate(
        [
            segment_table,
            ln_gamma.reshape(1, D),
            ln_beta.reshape(1, D),
            jnp.zeros((3, D), jnp.float32),
        ],
        axis=0,
    )
    return _ln_call(gathered, seg3, pos, params)


# speedup vs baseline: 1.3161x; 1.1206x over previous
import functools

import jax
import jax.numpy as jnp
from jax import lax
from jax.experimental import pallas as pl
from jax.experimental.pallas import tpu as pltpu
from jax.experimental.pallas import tpu_sc as plsc

B, L, V, D = 1024, 200, 1000000, 64
N = B * L
DP = 128


@functools.cache
def _make_sc_gather():
    info = plsc.get_sparse_core_info()
    NC, NS = info.num_cores, info.num_subcores
    NW = NC * NS
    per_w = N // NW
    CH = 800
    NCH = per_w // CH

    mesh = plsc.VectorSubcoreMesh(core_axis_name="c", subcore_axis_name="s")

    @functools.partial(
        pl.kernel,
        mesh=mesh,
        out_type=jax.ShapeDtypeStruct((N, DP), jnp.float32),
        scratch_types=[
            pltpu.VMEM((CH,), jnp.int32),
            pltpu.VMEM((CH, D), jnp.float32),
            pltpu.SemaphoreType.DMA,
        ],
        compiler_params=pltpu.CompilerParams(use_tc_tiling_on_sc=False),
    )
    def gather_kernel(idx_hbm, table_hbm, out_hbm, idx_v, rows_v, sem):
        wid = lax.axis_index("s") * NC + lax.axis_index("c")
        base = wid * per_w

        def body(c, carry):
            off = base + c * CH
            pltpu.sync_copy(idx_hbm.at[pl.ds(off, CH)], idx_v)
            pltpu.async_copy(table_hbm.at[idx_v], rows_v, sem).wait()
            pltpu.sync_copy(rows_v, out_hbm.at[pl.ds(off, CH), pl.ds(0, D)])
            return carry

        lax.fori_loop(0, NCH, body, 0)

    return gather_kernel


_BB = 32


def _ln_body(g_ref, m1_ref, m2_ref, pos_ref, par_ref, o_ref):
    x = g_ref[:, :, :D]
    pos = pos_ref[...]
    par = par_ref[...]

    s0 = par[0:1, :].reshape(1, 1, D)
    d1 = (par[1:2, :] - par[0:1, :]).reshape(1, 1, D)
    d2 = (par[2:3, :] - par[1:2, :]).reshape(1, 1, D)
    gam = par[3:4, :].reshape(1, 1, D)
    bet = par[4:5, :].reshape(1, 1, D)

    m1 = lax.broadcast_in_dim(m1_ref[...], (_BB, L, D), (0, 1))
    m2 = lax.broadcast_in_dim(m2_ref[...], (_BB, L, D), (0, 1))
    x = x + pos + s0 + m1 * d1 + m2 * d2

    mu = jnp.mean(x, axis=-1, keepdims=True)
    xc = x - mu
    var = jnp.mean(xc * xc, axis=-1, keepdims=True)
    o_ref[...] = xc * lax.rsqrt(var + 1e-6) * gam + bet


def _ln_call(gathered, m1f, m2f, pos, params):
    return pl.pallas_call(
        _ln_body,
        grid=(B // _BB,),
        in_specs=[
            pl.BlockSpec((_BB, L, DP), lambda i: (i, 0, 0)),
            pl.BlockSpec((_BB, L), lambda i: (i, 0)),
            pl.BlockSpec((_BB, L), lambda i: (i, 0)),
            pl.BlockSpec((1, L, D), lambda i: (0, 0, 0)),
            pl.BlockSpec((8, D), lambda i: (0, 0)),
        ],
        out_specs=pl.BlockSpec((_BB, L, D), lambda i: (i, 0, 0)),
        out_shape=jax.ShapeDtypeStruct((B, L, D), jnp.float32),
    )(gathered, m1f, m2f, pos, params)


def kernel(src, seg, word_table, position_table, segment_table, ln_gamma, ln_beta):
    idx = src.reshape(N).astype(jnp.int32)
    gathered = _make_sc_gather()(idx, word_table)
    gathered = gathered.reshape(B, L, DP)

    m1f = (seg >= 1).astype(jnp.float32)
    m2f = (seg >= 2).astype(jnp.float32)
    pos = position_table[:L].reshape(1, L, D)
    params = jnp.concatenate(
        [
            segment_table,
            ln_gamma.reshape(1, D),
            ln_beta.reshape(1, D),
            jnp.zeros((3, D), jnp.float32),
        ],
        axis=0,
    )
    return _ln_call(gathered, m1f, m2f, pos, params)


# speedup vs baseline: 1.4034x; 1.0663x over previous
import functools

import jax
import jax.numpy as jnp
from jax import lax
from jax.experimental import pallas as pl
from jax.experimental.pallas import tpu as pltpu
from jax.experimental.pallas import tpu_sc as plsc

B, L, V, D = 1024, 200, 1000000, 64
N = B * L
DP = 128


@functools.cache
def _make_sc_gather():
    info = plsc.get_sparse_core_info()
    NC, NS = info.num_cores, info.num_subcores
    NW = NC * NS
    per_w = N // NW
    CH = 800
    NCH = per_w // CH

    mesh = plsc.VectorSubcoreMesh(core_axis_name="c", subcore_axis_name="s")

    @functools.partial(
        pl.kernel,
        mesh=mesh,
        out_type=jax.ShapeDtypeStruct((N, DP), jnp.float32),
        scratch_types=[
            pltpu.VMEM((CH,), jnp.int32),
            pltpu.VMEM((CH, D), jnp.float32),
            pltpu.SemaphoreType.DMA,
        ],
        compiler_params=pltpu.CompilerParams(use_tc_tiling_on_sc=False),
    )
    def gather_kernel(idx_hbm, table_hbm, out_hbm, idx_v, rows_v, sem):
        wid = lax.axis_index("s") * NC + lax.axis_index("c")
        base = wid * per_w

        def body(c, carry):
            off = base + c * CH
            pltpu.sync_copy(idx_hbm.at[pl.ds(off, CH)], idx_v)
            pltpu.async_copy(table_hbm.at[idx_v], rows_v, sem).wait()
            pltpu.sync_copy(rows_v, out_hbm.at[pl.ds(off, CH), pl.ds(0, D)])
            return carry

        lax.fori_loop(0, NCH, body, 0)

    return gather_kernel


_LB = 8


def _ln_body(g_ref, m1_ref, m2_ref, pos_ref, par_ref, o_ref):
    x = g_ref[:, :, :D]
    pos = pos_ref[...]
    par = par_ref[...]

    s0 = par[0:1, :].reshape(1, 1, D)
    d1 = (par[1:2, :] - par[0:1, :]).reshape(1, 1, D)
    d2 = (par[2:3, :] - par[1:2, :]).reshape(1, 1, D)
    gam = par[3:4, :].reshape(1, 1, D)
    bet = par[4:5, :].reshape(1, 1, D)

    m1 = lax.broadcast_in_dim(m1_ref[...], (_LB, B, D), (0, 1))
    m2 = lax.broadcast_in_dim(m2_ref[...], (_LB, B, D), (0, 1))
    x = x + pos + s0 + m1 * d1 + m2 * d2

    mu = jnp.mean(x, axis=-1, keepdims=True)
    xc = x - mu
    var = jnp.mean(xc * xc, axis=-1, keepdims=True)
    y = xc * lax.rsqrt(var + 1e-6) * gam + bet
    o_ref[...] = jnp.swapaxes(y, 1, 2)


def _ln_call(gathered, m1t, m2t, pos, params):
    return pl.pallas_call(
        _ln_body,
        grid=(L // _LB,),
        in_specs=[
            pl.BlockSpec((_LB, B, DP), lambda i: (i, 0, 0)),
            pl.BlockSpec((_LB, B), lambda i: (i, 0)),
            pl.BlockSpec((_LB, B), lambda i: (i, 0)),
            pl.BlockSpec((_LB, 1, D), lambda i: (i, 0, 0)),
            pl.BlockSpec((8, D), lambda i: (0, 0)),
        ],
        out_specs=pl.BlockSpec((_LB, D, B), lambda i: (i, 0, 0)),
        out_shape=jax.ShapeDtypeStruct((L, D, B), jnp.float32),
    )(gathered, m1t, m2t, pos, params)


def kernel(src, seg, word_table, position_table, segment_table, ln_gamma, ln_beta):
    idx_t = src.astype(jnp.int32).T.reshape(N)
    gathered = _make_sc_gather()(idx_t, word_table)
    gathered = gathered.reshape(L, B, DP)

    seg_t = seg.astype(jnp.int32).T
    m1t = (seg_t >= 1).astype(jnp.float32)
    m2t = (seg_t >= 2).astype(jnp.float32)
    pos = position_table[:L].reshape(L, 1, D)
    params = jnp.concatenate(
        [
            segment_table,
            ln_gamma.reshape(1, D),
            ln_beta.reshape(1, D),
            jnp.zeros((3, D), jnp.float32),
        ],
        axis=0,
    )
    out_t = _ln_call(gathered, m1t, m2t, pos, params)
    return out_t.transpose(2, 0, 1)


# speedup vs baseline: 1.4101x; 1.0048x over previous
import functools

import jax
import jax.numpy as jnp
from jax import lax
from jax.experimental import pallas as pl
from jax.experimental.pallas import tpu as pltpu
from jax.experimental.pallas import tpu_sc as plsc

B, L, V, D = 1024, 200, 1000000, 64
N = B * L
DP = 128


@functools.cache
def _make_sc_gather():
    info = plsc.get_sparse_core_info()
    NC, NS = info.num_cores, info.num_subcores
    NW = NC * NS
    per_w = N // NW
    CH = 800
    NCH = per_w // CH

    mesh = plsc.VectorSubcoreMesh(core_axis_name="c", subcore_axis_name="s")

    @functools.partial(
        pl.kernel,
        mesh=mesh,
        out_type=jax.ShapeDtypeStruct((N, DP), jnp.float32),
        scratch_types=[
            pltpu.VMEM((per_w,), jnp.int32),
            pltpu.VMEM((CH, D), jnp.float32),
            pltpu.VMEM((CH, D), jnp.float32),
            pltpu.SemaphoreType.DMA,
            pltpu.SemaphoreType.DMA,
            pltpu.SemaphoreType.DMA,
            pltpu.SemaphoreType.DMA,
        ],
        compiler_params=pltpu.CompilerParams(use_tc_tiling_on_sc=False),
    )
    def gather_kernel(idx_hbm, table_hbm, out_hbm, idx_v, rows_a, rows_b,
                      gsem_a, gsem_b, wsem_a, wsem_b):
        wid = lax.axis_index("s") * NC + lax.axis_index("c")
        base = wid * per_w

        pltpu.sync_copy(idx_hbm.at[pl.ds(base, per_w)], idx_v)

        rows = (rows_a, rows_b)
        gsem = (gsem_a, gsem_b)
        wsem = (wsem_a, wsem_b)
        wb = [None, None]
        for c in range(NCH):
            k = c % 2
            if wb[k] is not None:
                wb[k].wait()
            g = pltpu.async_copy(
                table_hbm.at[idx_v.at[pl.ds(c * CH, CH)]], rows[k], gsem[k])
            g.wait()
            wb[k] = pltpu.async_copy(
                rows[k], out_hbm.at[pl.ds(base + c * CH, CH), pl.ds(0, D)],
                wsem[k])
        wb[0].wait()
        wb[1].wait()

    return gather_kernel


_LB = 8


def _ln_body(g_ref, m1_ref, m2_ref, pos_ref, par_ref, o_ref):
    x = g_ref[:, :, :D]
    pos = pos_ref[...]
    par = par_ref[...]

    s0 = par[0:1, :].reshape(1, 1, D)
    d1 = (par[1:2, :] - par[0:1, :]).reshape(1, 1, D)
    d2 = (par[2:3, :] - par[1:2, :]).reshape(1, 1, D)
    gam = par[3:4, :].reshape(1, 1, D)
    bet = par[4:5, :].reshape(1, 1, D)

    m1 = lax.broadcast_in_dim(m1_ref[...], (_LB, B, D), (0, 1))
    m2 = lax.broadcast_in_dim(m2_ref[...], (_LB, B, D), (0, 1))
    x = x + pos + s0 + m1 * d1 + m2 * d2

    mu = jnp.mean(x, axis=-1, keepdims=True)
    xc = x - mu
    var = jnp.mean(xc * xc, axis=-1, keepdims=True)
    y = xc * lax.rsqrt(var + 1e-6) * gam + bet
    o_ref[...] = jnp.swapaxes(y, 1, 2)


def _ln_call(gathered, m1t, m2t, pos, params):
    return pl.pallas_call(
        _ln_body,
        grid=(L // _LB,),
        in_specs=[
            pl.BlockSpec((_LB, B, DP), lambda i: (i, 0, 0)),
            pl.BlockSpec((_LB, B), lambda i: (i, 0)),
            pl.BlockSpec((_LB, B), lambda i: (i, 0)),
            pl.BlockSpec((_LB, 1, D), lambda i: (i, 0, 0)),
            pl.BlockSpec((8, D), lambda i: (0, 0)),
        ],
        out_specs=pl.BlockSpec((_LB, D, B), lambda i: (i, 0, 0)),
        out_shape=jax.ShapeDtypeStruct((L, D, B), jnp.float32),
        compiler_params=pltpu.CompilerParams(vmem_limit_bytes=100 * 1024 * 1024),
    )(gathered, m1t, m2t, pos, params)


def kernel(src, seg, word_table, position_table, segment_table, ln_gamma, ln_beta):
    idx_t = src.astype(jnp.int32).T.reshape(N)
    gathered = _make_sc_gather()(idx_t, word_table)
    gathered = gathered.reshape(L, B, DP)

    seg_t = seg.astype(jnp.int32).T
    m1t = (seg_t >= 1).astype(jnp.float32)
    m2t = (seg_t >= 2).astype(jnp.float32)
    pos = position_table[:L].reshape(L, 1, D)
    params = jnp.concatenate(
        [
            segment_table,
            ln_gamma.reshape(1, D),
            ln_beta.reshape(1, D),
            jnp.zeros((3, D), jnp.float32),
        ],
        axis=0,
    )
    out_t = _ln_call(gathered, m1t, m2t, pos, params)
    return out_t.transpose(2, 0, 1)


# speedup vs baseline: 2.3299x; 1.6523x over previous
import functools

import jax
import jax.numpy as jnp
from jax import lax
from jax.experimental import pallas as pl
from jax.experimental.pallas import tpu as pltpu
from jax.experimental.pallas import tpu_sc as plsc

B, L, V, D = 1024, 200, 1000000, 64
N = B * L
DP = 128


@functools.cache
def _make_sc_gather():
    info = plsc.get_sparse_core_info()
    NC, NS = info.num_cores, info.num_subcores
    NW = NC * NS
    per_w = N // NW
    CH = 400
    NCH = per_w // CH

    mesh = plsc.VectorSubcoreMesh(core_axis_name="c", subcore_axis_name="s")

    @functools.partial(
        pl.kernel,
        mesh=mesh,
        out_type=jax.ShapeDtypeStruct((N, DP), jnp.float32),
        scratch_types=[
            pltpu.VMEM((per_w,), jnp.int32),
            pltpu.VMEM((CH, DP), jnp.float32),
            pltpu.VMEM((CH, DP), jnp.float32),
            pltpu.SemaphoreType.DMA,
            pltpu.SemaphoreType.DMA,
            pltpu.SemaphoreType.DMA,
            pltpu.SemaphoreType.DMA,
        ],
        compiler_params=pltpu.CompilerParams(use_tc_tiling_on_sc=False),
    )
    def gather_kernel(idx_hbm, table_hbm, out_hbm, idx_v, rows_a, rows_b,
                      gsem_a, gsem_b, wsem_a, wsem_b):
        wid = lax.axis_index("s") * NC + lax.axis_index("c")
        base = wid * per_w

        pltpu.sync_copy(idx_hbm.at[pl.ds(base, per_w)], idx_v)

        rows = (rows_a, rows_b)
        gsem = (gsem_a, gsem_b)
        wsem = (wsem_a, wsem_b)
        wb = [None, None]
        for c in range(NCH):
            k = c % 2
            if wb[k] is not None:
                wb[k].wait()
            g = pltpu.async_copy(
                table_hbm.at[idx_v.at[pl.ds(c * CH, CH)]], rows[k], gsem[k])
            g.wait()
            wb[k] = pltpu.async_copy(
                rows[k], out_hbm.at[pl.ds(base + c * CH, CH)], wsem[k])
        wb[0].wait()
        wb[1].wait()

    return gather_kernel


_RB = 16384


def _fmt_body(t_ref, o_ref):
    o_ref[:, :D] = jnp.swapaxes(t_ref[...], 0, 1)


def _fmt_call(wt_t):
    nb = (V + _RB - 1) // _RB
    return pl.pallas_call(
        _fmt_body,
        grid=(nb,),
        in_specs=[pl.BlockSpec((D, _RB), lambda i: (0, i))],
        out_specs=pl.BlockSpec((_RB, DP), lambda i: (i, 0)),
        out_shape=jax.ShapeDtypeStruct((V, DP), jnp.float32),
        compiler_params=pltpu.CompilerParams(vmem_limit_bytes=100 * 1024 * 1024),
    )(wt_t)


_LB = 8


def _ln_body(g_ref, m1_ref, m2_ref, pos_ref, par_ref, o_ref):
    x = g_ref[:, :, :D]
    pos = pos_ref[...]
    par = par_ref[...]

    s0 = par[0:1, :].reshape(1, 1, D)
    d1 = (par[1:2, :] - par[0:1, :]).reshape(1, 1, D)
    d2 = (par[2:3, :] - par[1:2, :]).reshape(1, 1, D)
    gam = par[3:4, :].reshape(1, 1, D)
    bet = par[4:5, :].reshape(1, 1, D)

    m1 = lax.broadcast_in_dim(m1_ref[...], (_LB, B, D), (0, 1))
    m2 = lax.broadcast_in_dim(m2_ref[...], (_LB, B, D), (0, 1))
    x = x + pos + s0 + m1 * d1 + m2 * d2

    mu = jnp.mean(x, axis=-1, keepdims=True)
    xc = x - mu
    var = jnp.mean(xc * xc, axis=-1, keepdims=True)
    y = xc * lax.rsqrt(var + 1e-6) * gam + bet
    o_ref[...] = jnp.swapaxes(y, 1, 2)


def _ln_call(gathered, m1t, m2t, pos, params):
    return pl.pallas_call(
        _ln_body,
        grid=(L // _LB,),
        in_specs=[
            pl.BlockSpec((_LB, B, DP), lambda i: (i, 0, 0)),
            pl.BlockSpec((_LB, B), lambda i: (i, 0)),
            pl.BlockSpec((_LB, B), lambda i: (i, 0)),
            pl.BlockSpec((_LB, 1, D), lambda i: (i, 0, 0)),
            pl.BlockSpec((8, D), lambda i: (0, 0)),
        ],
        out_specs=pl.BlockSpec((_LB, D, B), lambda i: (i, 0, 0)),
        out_shape=jax.ShapeDtypeStruct((L, D, B), jnp.float32),
        compiler_params=pltpu.CompilerParams(vmem_limit_bytes=100 * 1024 * 1024),
    )(gathered, m1t, m2t, pos, params)


def kernel(src, seg, word_table, position_table, segment_table, ln_gamma, ln_beta):
    idx_t = src.astype(jnp.int32).T.reshape(N)
    wt_pad = _fmt_call(word_table.T)
    gathered = _make_sc_gather()(idx_t, wt_pad)
    gathered = gathered.reshape(L, B, DP)

    seg_t = seg.astype(jnp.int32).T
    m1t = (seg_t >= 1).astype(jnp.float32)
    m2t = (seg_t >= 2).astype(jnp.float32)
    pos = position_table[:L].reshape(L, 1, D)
    params = jnp.concatenate(
        [
            segment_table,
            ln_gamma.reshape(1, D),
            ln_beta.reshape(1, D),
            jnp.zeros((3, D), jnp.float32),
        ],
        axis=0,
    )
    out_t = _ln_call(gathered, m1t, m2t, pos, params)
    return out_t.transpose(2, 0, 1)


# speedup vs baseline: 2.4615x; 1.0565x over previous
import functools

import jax
import jax.numpy as jnp
from jax import lax
from jax.experimental import pallas as pl
from jax.experimental.pallas import tpu as pltpu
from jax.experimental.pallas import tpu_sc as plsc

B, L, V, D = 1024, 200, 1000000, 64
N = B * L
DP = 128


@functools.cache
def _make_sc_gather():
    info = plsc.get_sparse_core_info()
    NC, NS = info.num_cores, info.num_subcores
    NW = NC * NS
    per_w = N // NW
    CH = 400
    NCH = per_w // CH

    mesh = plsc.VectorSubcoreMesh(core_axis_name="c", subcore_axis_name="s")

    @functools.partial(
        pl.kernel,
        mesh=mesh,
        out_type=jax.ShapeDtypeStruct((N, DP), jnp.float32),
        scratch_types=[
            pltpu.VMEM((per_w,), jnp.int32),
            pltpu.VMEM((CH, DP), jnp.float32),
            pltpu.VMEM((CH, DP), jnp.float32),
            pltpu.SemaphoreType.DMA,
            pltpu.SemaphoreType.DMA,
            pltpu.SemaphoreType.DMA,
            pltpu.SemaphoreType.DMA,
        ],
        compiler_params=pltpu.CompilerParams(use_tc_tiling_on_sc=False),
    )
    def gather_kernel(idx_hbm, table_hbm, out_hbm, idx_v, rows_a, rows_b,
                      gsem_a, gsem_b, wsem_a, wsem_b):
        wid = lax.axis_index("s") * NC + lax.axis_index("c")
        base = wid * per_w

        pltpu.sync_copy(idx_hbm.at[pl.ds(base, per_w)], idx_v)

        rows = (rows_a, rows_b)
        gsem = (gsem_a, gsem_b)
        wsem = (wsem_a, wsem_b)
        wb = [None, None]
        for c in range(NCH):
            k = c % 2
            if wb[k] is not None:
                wb[k].wait()
            g = pltpu.async_copy(
                table_hbm.at[idx_v.at[pl.ds(c * CH, CH)]], rows[k], gsem[k])
            g.wait()
            wb[k] = pltpu.async_copy(
                rows[k], out_hbm.at[pl.ds(base + c * CH, CH)], wsem[k])
        wb[0].wait()
        wb[1].wait()

    return gather_kernel


_RB = 16384


def _fmt_body(t_ref, o_ref):
    o_ref[:, :D] = jnp.swapaxes(t_ref[...], 0, 1)


def _fmt_call(wt_t):
    nb = (V + _RB - 1) // _RB
    return pl.pallas_call(
        _fmt_body,
        grid=(nb,),
        in_specs=[pl.BlockSpec((D, _RB), lambda i: (0, i))],
        out_specs=pl.BlockSpec((_RB, DP), lambda i: (i, 0)),
        out_shape=jax.ShapeDtypeStruct((V, DP), jnp.float32),
        compiler_params=pltpu.CompilerParams(vmem_limit_bytes=100 * 1024 * 1024),
    )(wt_t)


_LB = 8


def _ln_body(g_ref, sg_ref, pos_ref, par_ref, o_ref):
    x = g_ref[:, :, :D]
    pos = pos_ref[...]
    par = par_ref[...]

    s0 = par[0:1, :].reshape(1, 1, D)
    d1 = (par[1:2, :] - par[0:1, :]).reshape(1, 1, D)
    d2 = (par[2:3, :] - par[1:2, :]).reshape(1, 1, D)
    gam = par[3:4, :].reshape(1, 1, D)
    bet = par[4:5, :].reshape(1, 1, D)

    sg = lax.broadcast_in_dim(sg_ref[...], (_LB, B, D), (0, 1))
    x = x + pos + s0 + jnp.minimum(sg, 1.0) * d1 + jnp.maximum(sg - 1.0, 0.0) * d2

    mu = jnp.mean(x, axis=-1, keepdims=True)
    xc = x - mu
    var = jnp.mean(xc * xc, axis=-1, keepdims=True)
    y = xc * lax.rsqrt(var + 1e-6) * gam + bet
    o_ref[...] = jnp.swapaxes(y, 1, 2)


def _ln_call(gathered, sgf, pos, params):
    return pl.pallas_call(
        _ln_body,
        grid=(L // _LB,),
        in_specs=[
            pl.BlockSpec((_LB, B, DP), lambda i: (i, 0, 0)),
            pl.BlockSpec((_LB, B), lambda i: (i, 0)),
            pl.BlockSpec((_LB, 1, D), lambda i: (i, 0, 0)),
            pl.BlockSpec((8, D), lambda i: (0, 0)),
        ],
        out_specs=pl.BlockSpec((_LB, D, B), lambda i: (i, 0, 0)),
        out_shape=jax.ShapeDtypeStruct((L, D, B), jnp.float32),
        compiler_params=pltpu.CompilerParams(vmem_limit_bytes=100 * 1024 * 1024),
    )(gathered, sgf, pos, params)


def kernel(src, seg, word_table, position_table, segment_table, ln_gamma, ln_beta):
    idx_t = src.astype(jnp.int32).T.reshape(N)
    wt_pad = _fmt_call(word_table.T)
    gathered = _make_sc_gather()(idx_t, wt_pad)
    gathered = gathered.reshape(L, B, DP)

    sgf = seg.astype(jnp.float32).T
    pos = position_table[:L].reshape(L, 1, D)
    params = jnp.concatenate(
        [
            segment_table,
            ln_gamma.reshape(1, D),
            ln_beta.reshape(1, D),
            jnp.zeros((3, D), jnp.float32),
        ],
        axis=0,
    )
    out_t = _ln_call(gathered, sgf, pos, params)
    return out_t.transpose(2, 0, 1)
